# baseline (speedup 1.0000x reference)
import functools

import jax
import jax.numpy as jnp
from jax import lax
from jax.experimental import pallas as pl
from jax.experimental.pallas import tpu as pltpu
from jax.experimental.pallas import tpu_sc as plsc

VOCAB = 100000
HIDDEN = 768
MAX_POS = 8192
BATCH = 4
SEQ = 4096
EPS = 1e-5

L = 16
NC, NS = 2, 16
NW = NC * NS
NTOK = BATCH * SEQ
TOK_PER_W = NTOK // NW
T = 16
NCHUNK = TOK_PER_W // T
NGRP = NCHUNK // BATCH
NCH = HIDDEN // L
CG = 8
NCG = NCH // CG

_GDN = lax.GatherDimensionNumbers(
    offset_dims=(), collapsed_slice_dims=(0,), start_index_map=(0,))


def _lane_sum(v):
    for sh in (8, 4, 2, 1):
        idx = (jnp.arange(L, dtype=jnp.int32) + sh) % L
        perm = lax.gather(v, idx[:, None], _GDN, (1,),
                          mode=lax.GatherScatterMode.PROMISE_IN_BOUNDS)
        v = v + perm
    return v


def _rsqrt_vec(v):
    bits = lax.bitcast_convert_type(v, jnp.int32)
    y = lax.bitcast_convert_type(jnp.int32(0x5F3759DF) - (bits >> 1),
                                 jnp.float32)
    for _ in range(2):
        y = y * (1.5 - 0.5 * v * y * y)
    return y


@functools.partial(
    pl.kernel,
    mesh=plsc.VectorSubcoreMesh(core_axis_name="c", subcore_axis_name="s"),
    out_type=jax.ShapeDtypeStruct((NTOK, HIDDEN), jnp.float32),
    scratch_types=[
        pltpu.VMEM((TOK_PER_W,), jnp.int32),
        pltpu.VMEM((4, T, HIDDEN), jnp.float32),
        pltpu.VMEM((T, HIDDEN), jnp.float32),
        pltpu.VMEM((HIDDEN,), jnp.float32),
        pltpu.VMEM((HIDDEN,), jnp.float32),
        pltpu.VMEM((4, T * L), jnp.float32),
        pltpu.VMEM((4, T * L), jnp.float32),
        pltpu.SemaphoreType.DMA,
        pltpu.SemaphoreType.DMA,
        pltpu.SemaphoreType.DMA,
        pltpu.SemaphoreType.DMA,
        pltpu.SemaphoreType.DMA,
        pltpu.SemaphoreType.DMA,
        pltpu.SemaphoreType.DMA,
        pltpu.SemaphoreType.DMA,
        pltpu.SemaphoreType.DMA,
    ],
)
def _embed_ln_kernel(x_hbm, tok_tbl, pos_tbl, gam_hbm, bet_hbm, out_hbm,
                     idx_v, rows2, pos_v, gam_v, bet_v, ps_v, pq_v,
                     semg0, semg1, semg2, semg3, semp,
                     semw0, semw1, semw2, semw3):
    wid = lax.axis_index("s") * NC + lax.axis_index("c")
    w0 = wid * (SEQ // NW)
    pltpu.sync_copy(gam_hbm, gam_v)
    pltpu.sync_copy(bet_hbm, bet_v)
    for b in range(BATCH):
        pltpu.sync_copy(x_hbm.at[pl.ds(b * SEQ + w0, SEQ // NW)],
                        idx_v.at[pl.ds(b * (SEQ // NW), SEQ // NW)])
    semg = (semg0, semg1, semg2, semg3)
    semw = (semw0, semw1, semw2, semw3)

    def idx_off(ch):
        jn = ch // BATCH
        bn = lax.rem(ch, BATCH)
        return bn * (SEQ // NW) + jn * T

    def out_base(ch):
        jn = ch // BATCH
        bn = lax.rem(ch, BATCH)
        return bn * SEQ + w0 + jn * T

    def gather_in(ch, slot):
        pltpu.async_copy(tok_tbl.at[idx_v.at[pl.ds(idx_off(ch), T)]],
                         rows2.at[slot], semg[slot])

    def wait_gather(ch, slot):
        pltpu.make_async_copy(tok_tbl.at[idx_v.at[pl.ds(idx_off(ch), T)]],
                              rows2.at[slot], semg[slot]).wait()

    def pos_in(j):
        pltpu.async_copy(pos_tbl.at[pl.ds(w0 + j * T, T), :], pos_v, semp)

    def wait_pos(j):
        pltpu.make_async_copy(pos_tbl.at[pl.ds(w0 + j * T, T), :],
                              pos_v, semp).wait()

    def writeout(ch, slot):
        pltpu.async_copy(rows2.at[slot],
                         out_hbm.at[pl.ds(out_base(ch), T), :], semw[slot])

    def wait_writeout(ch, slot):
        pltpu.make_async_copy(rows2.at[slot],
                              out_hbm.at[pl.ds(out_base(ch), T), :],
                              semw[slot]).wait()

    for b in range(BATCH):
        gather_in(b, b)
    pos_in(0)

    def compute_group(j, carry):
        for b in range(BATCH):
            wait_gather(BATCH * j + b, b)
        wait_pos(j)

        def make_half(c0, b0, first):
            bpair = (b0, b0 + 1)

            @plsc.parallel_loop(0, T)
            def half(t):
                accs = {b: jnp.zeros((L,), jnp.float32) for b in bpair}
                accq = {b: jnp.zeros((L,), jnp.float32) for b in bpair}
                for c in range(c0, c0 + NCH // 2):
                    pv = pos_v[t, pl.ds(c * L, L)]
                    for b in bpair:
                        v = rows2[b, t, pl.ds(c * L, L)] + pv
                        rows2[b, t, pl.ds(c * L, L)] = v
                        accs[b] = accs[b] + v
                        accq[b] = accq[b] + v * v
                for b in bpair:
                    if first:
                        ps_v[b, pl.ds(t * L, L)] = accs[b]
                        pq_v[b, pl.ds(t * L, L)] = accq[b]
                    else:
                        ps_v[b, pl.ds(t * L, L)] = (ps_v[b, pl.ds(t * L, L)]
                                                    + accs[b])
                        pq_v[b, pl.ds(t * L, L)] = (pq_v[b, pl.ds(t * L, L)]
                                                    + accq[b])

        make_half(0, 0, True)
        make_half(0, 2, True)
        make_half(NCH // 2, 0, False)
        make_half(NCH // 2, 2, False)

        @pl.when(j < NGRP - 1)
        def _():
            pos_in(j + 1)

        for b in range(BATCH):
            ch = BATCH * j + b

            @plsc.parallel_loop(0, T)
            def phase_s(t):
                s_tot = ps_v[b, pl.ds(t * L, L)]
                q_tot = pq_v[b, pl.ds(t * L, L)]
                meanv = _lane_sum(s_tot) * (1.0 / HIDDEN)
                varv = _lane_sum(q_tot) * (1.0 / HIDDEN) - meanv * meanv
                rstdv = _rsqrt_vec(varv + EPS)
                ps_v[b, pl.ds(t * L, L)] = rstdv
                pq_v[b, pl.ds(t * L, L)] = meanv * rstdv

            for cg in range(NCG):
                gs = [gam_v[pl.ds((cg * CG + j2) * L, L)] for j2 in range(CG)]
                bs = [bet_v[pl.ds((cg * CG + j2) * L, L)] for j2 in range(CG)]

                @plsc.parallel_loop(0, T)
                def phase_b(t):
                    p = ps_v[b, pl.ds(t * L, L)]
                    q = pq_v[b, pl.ds(t * L, L)]
                    for j2 in range(CG):
                        c = cg * CG + j2
                        v = rows2[b, t, pl.ds(c * L, L)]
                        rows2[b, t, pl.ds(c * L, L)] = ((v * p - q) * gs[j2]
                                                        + bs[j2])

            writeout(ch, b)

            if b >= 1:
                @pl.when(j < NGRP - 1)
                def _():
                    wait_writeout(BATCH * j + b - 1, b - 1)
                    gather_in(BATCH * (j + 1) + b - 1, b - 1)

        @pl.when(j < NGRP - 1)
        def _():
            wait_writeout(BATCH * j + BATCH - 1, BATCH - 1)
            gather_in(BATCH * (j + 1) + BATCH - 1, BATCH - 1)

        return carry

    lax.fori_loop(0, NGRP, compute_group, 0)
    for b in range(BATCH):
        wait_writeout(NCHUNK - BATCH + b, b)


def kernel(x, token_table, pos_table, ln_gamma, ln_beta):
    x_flat = x.reshape(-1).astype(jnp.int32)
    out = _embed_ln_kernel(x_flat, token_table, pos_table, ln_gamma, ln_beta)
    return out.reshape(BATCH, SEQ, HIDDEN)


# speedup vs baseline: 1.5816x; 1.5816x over previous
import functools

import jax
import jax.numpy as jnp
from jax import lax
from jax.experimental import pallas as pl
from jax.experimental.pallas import tpu as pltpu
from jax.experimental.pallas import tpu_sc as plsc

VOCAB = 100000
HIDDEN = 768
MAX_POS = 8192
BATCH = 4
SEQ = 4096
EPS = 1e-5

L = 16
NC, NS = 2, 16
NW = NC * NS
NTOK = BATCH * SEQ
TOK_PER_W = NTOK // NW
T = 32
NCHUNK = TOK_PER_W // T
NCH = HIDDEN // L
CG = 16
NCG = NCH // CG

_GDN = lax.GatherDimensionNumbers(
    offset_dims=(), collapsed_slice_dims=(0,), start_index_map=(0,))


def _lane_sum(v):
    for sh in (8, 4, 2, 1):
        idx = (jnp.arange(L, dtype=jnp.int32) + sh) % L
        perm = lax.gather(v, idx[:, None], _GDN, (1,),
                          mode=lax.GatherScatterMode.PROMISE_IN_BOUNDS)
        v = v + perm
    return v


def _rsqrt_vec(v):
    bits = lax.bitcast_convert_type(v, jnp.int32)
    y = lax.bitcast_convert_type(jnp.int32(0x5F3759DF) - (bits >> 1),
                                 jnp.float32)
    for _ in range(2):
        y = y * (1.5 - 0.5 * v * y * y)
    return y


@functools.partial(
    pl.kernel,
    mesh=plsc.VectorSubcoreMesh(core_axis_name="c", subcore_axis_name="s"),
    out_type=jax.ShapeDtypeStruct((NTOK, HIDDEN), jnp.float32),
    scratch_types=[
        pltpu.VMEM((TOK_PER_W,), jnp.int32),
        pltpu.VMEM((4, T, HIDDEN), jnp.float32),
        pltpu.VMEM((T, HIDDEN), jnp.float32),
        pltpu.VMEM((HIDDEN,), jnp.float32),
        pltpu.VMEM((HIDDEN,), jnp.float32),
        pltpu.VMEM((T * L,), jnp.float32),
        pltpu.VMEM((T * L,), jnp.float32),
        pltpu.VMEM((T * L,), jnp.float32),
        pltpu.VMEM((T * L,), jnp.float32),
        pltpu.VMEM((T * L,), jnp.float32),
        pltpu.VMEM((T * L,), jnp.float32),
        pltpu.SemaphoreType.DMA,
        pltpu.SemaphoreType.DMA,
        pltpu.SemaphoreType.DMA,
        pltpu.SemaphoreType.DMA,
        pltpu.SemaphoreType.DMA,
        pltpu.SemaphoreType.DMA,
        pltpu.SemaphoreType.DMA,
        pltpu.SemaphoreType.DMA,
        pltpu.SemaphoreType.DMA,
    ],
)
def _embed_ln_kernel(x_hbm, tok_tbl, pos_tbl, gam_hbm, bet_hbm, out_hbm,
                     idx_v, rows2, pos_v, gam_v, bet_v,
                     sum_v, sq_v, sum2_v, sq2_v, p_v, q_v,
                     semg0, semg1, semg2, semg3, semp,
                     semw0, semw1, semw2, semw3):
    wid = lax.axis_index("s") * NC + lax.axis_index("c")
    w0 = wid * (SEQ // NW)
    pltpu.sync_copy(gam_hbm, gam_v)
    pltpu.sync_copy(bet_hbm, bet_v)
    for b in range(BATCH):
        pltpu.sync_copy(x_hbm.at[pl.ds(b * SEQ + w0, SEQ // NW)],
                        idx_v.at[pl.ds(b * (SEQ // NW), SEQ // NW)])
    semg = (semg0, semg1, semg2, semg3)
    semw = (semw0, semw1, semw2, semw3)

    def idx_off(ch):
        jn = ch // BATCH
        bn = lax.rem(ch, BATCH)
        return bn * (SEQ // NW) + jn * T

    def out_base(ch):
        jn = ch // BATCH
        bn = lax.rem(ch, BATCH)
        return bn * SEQ + w0 + jn * T

    def gather_in(ch, slot):
        pltpu.async_copy(tok_tbl.at[idx_v.at[pl.ds(idx_off(ch), T)]],
                         rows2.at[slot], semg[slot])

    def wait_gather(ch, slot):
        pltpu.make_async_copy(tok_tbl.at[idx_v.at[pl.ds(idx_off(ch), T)]],
                              rows2.at[slot], semg[slot]).wait()

    def pos_in(j):
        pltpu.async_copy(pos_tbl.at[pl.ds(w0 + j * T, T), :], pos_v, semp)

    def wait_pos(j):
        pltpu.make_async_copy(pos_tbl.at[pl.ds(w0 + j * T, T), :],
                              pos_v, semp).wait()

    def writeout(ch, slot):
        pltpu.async_copy(rows2.at[slot],
                         out_hbm.at[pl.ds(out_base(ch), T), :], semw[slot])

    def wait_writeout(ch, slot):
        pltpu.make_async_copy(rows2.at[slot],
                              out_hbm.at[pl.ds(out_base(ch), T), :],
                              semw[slot]).wait()

    gather_in(0, 0)
    gather_in(1, 1)
    pos_in(0)

    def compute_chunk(j, b):
        slot = b
        ch = BATCH * j + b
        rows_v = rows2.at[slot]

        nslot = (b + 2) % BATCH
        @pl.when(ch + 2 <= NCHUNK - 1)
        def _():
            @pl.when(ch >= 2)
            def _():
                wait_writeout(ch - 2, nslot)
            gather_in(ch + 2, nslot)

        wait_gather(ch, slot)
        if b == 0:
            wait_pos(j)

        def make_half(c0, s_ref, q_ref):
            @plsc.parallel_loop(0, T)
            def half(t):
                accs = [jnp.zeros((L,), jnp.float32) for _ in range(2)]
                accq = [jnp.zeros((L,), jnp.float32) for _ in range(2)]
                for c in range(c0, c0 + NCH // 2):
                    v = rows_v[t, pl.ds(c * L, L)] + pos_v[t, pl.ds(c * L, L)]
                    rows_v[t, pl.ds(c * L, L)] = v
                    accs[c % 2] = accs[c % 2] + v
                    accq[c % 2] = accq[c % 2] + v * v
                s_ref[pl.ds(t * L, L)] = accs[0] + accs[1]
                q_ref[pl.ds(t * L, L)] = accq[0] + accq[1]

        make_half(0, sum_v, sq_v)
        make_half(NCH // 2, sum2_v, sq2_v)

        if b == BATCH - 1:
            @pl.when(j < NCHUNK // BATCH - 1)
            def _():
                pos_in(j + 1)

        @plsc.parallel_loop(0, T)
        def phase_s(t):
            s_tot = sum_v[pl.ds(t * L, L)] + sum2_v[pl.ds(t * L, L)]
            q_tot = sq_v[pl.ds(t * L, L)] + sq2_v[pl.ds(t * L, L)]
            meanv = _lane_sum(s_tot) * (1.0 / HIDDEN)
            varv = _lane_sum(q_tot) * (1.0 / HIDDEN) - meanv * meanv
            rstdv = _rsqrt_vec(varv + EPS)
            p_v[pl.ds(t * L, L)] = rstdv
            q_v[pl.ds(t * L, L)] = meanv * rstdv

        for cg in range(NCG):
            gs = [gam_v[pl.ds((cg * CG + j2) * L, L)] for j2 in range(CG)]
            bs = [bet_v[pl.ds((cg * CG + j2) * L, L)] for j2 in range(CG)]

            @plsc.parallel_loop(0, T)
            def phase_b(t):
                p = p_v[pl.ds(t * L, L)]
                q = q_v[pl.ds(t * L, L)]
                for j2 in range(CG):
                    c = cg * CG + j2
                    v = rows_v[t, pl.ds(c * L, L)]
                    rows_v[t, pl.ds(c * L, L)] = (v * p - q) * gs[j2] + bs[j2]

        writeout(ch, slot)

    def j_body(j, carry):
        for b in range(BATCH):
            compute_chunk(j, b)
        return carry

    lax.fori_loop(0, NCHUNK // BATCH, j_body, 0)
    for b in range(BATCH):
        wait_writeout(NCHUNK - BATCH + b, b)


def kernel(x, token_table, pos_table, ln_gamma, ln_beta):
    x_flat = x.reshape(-1).astype(jnp.int32)
    out = _embed_ln_kernel(x_flat, token_table, pos_table, ln_gamma, ln_beta)
    return out.reshape(BATCH, SEQ, HIDDEN)
